# baseline (speedup 1.0000x reference)
import functools

import jax
import jax.numpy as jnp
from jax import lax
from jax.experimental import pallas as pl
from jax.experimental.pallas import tpu as pltpu
from jax.experimental.pallas import tpu_sc as plsc

V = 512
D = 128
B = 8
T = 256
N = B * T
TOPK = 3
GAMMA = 0.25

SC_NC = 2
SC_NS = 16
SC_NW = SC_NC * SC_NS
GB_PER_W = N // SC_NW


def _fused_body(x_ref, cw_ref, gw_ref, gb_ref, emb_ref,
                probs_ref, vsel_ref, counts_ref, cpx_ref, loss_ref):
    b = pl.program_id(0)
    xb = x_ref[...]
    ze = lax.dot_general(xb, cw_ref[...], (((1,), (1,)), ((), ())),
                         preferred_element_type=jnp.float32)
    mean = jnp.mean(ze)
    var = jnp.mean((ze - mean) ** 2)
    zn = (ze - mean) * lax.rsqrt(var + 1e-5)
    zn = zn * gw_ref[...] + gb_ref[...]

    emb = emb_ref[...]
    g = lax.dot_general(zn, emb, (((1,), (1,)), ((), ())),
                        preferred_element_type=jnp.float32)
    e2 = emb * emb
    esq = lax.dot_general(jnp.ones((1, D), jnp.float32), e2,
                          (((1,), (1,)), ((), ())),
                          preferred_element_type=jnp.float32)
    zsq = jnp.sum(zn * zn, axis=1, keepdims=True)
    d2 = jnp.maximum(zsq - 2.0 * g + esq, 0.0)

    dmm = jnp.sqrt(d2)
    m = jnp.max(-dmm, axis=1, keepdims=True)
    e = jnp.exp(-dmm - m)
    probs_ref[...] = e / jnp.sum(e, axis=1, keepdims=True)

    iota = lax.broadcasted_iota(jnp.int32, (T, V), 1)
    cur = d2
    picks = []
    for _ in range(TOPK):
        mn = jnp.min(cur, axis=1, keepdims=True)
        ik = jnp.min(jnp.where(cur == mn, iota, 1 << 20), axis=1,
                     keepdims=True)
        cur = jnp.where(iota == ik, 1e30, cur)
        picks.append(ik)

    iota3 = lax.broadcasted_iota(jnp.int32, (TOPK * T, V), 1)
    ik3 = jnp.concatenate(picks, axis=0)
    oh3 = jnp.where(iota3 == ik3, 1.0, 0.0)
    rows3 = lax.dot_general(oh3, emb, (((1,), (0,)), ((), ())),
                            precision=lax.Precision.HIGHEST,
                            preferred_element_type=jnp.float32)
    zn3 = jnp.concatenate([zn] * TOPK, axis=0)
    diff3 = zn3 - rows3
    s3 = jnp.sum(diff3 * diff3, axis=1, keepdims=True)
    dd3 = jnp.sqrt(s3)
    rows = [rows3[k * T:(k + 1) * T] for k in range(TOPK)]
    d4 = jnp.concatenate([dd3[k * T:(k + 1) * T] for k in range(TOPK)],
                         axis=1)
    v4 = jnp.concatenate(picks, axis=1)
    best = jnp.min(d4, axis=1, keepdims=True)
    vsel = jnp.min(jnp.where(d4 == best, v4, 1 << 20), axis=1,
                   keepdims=True)
    vsel_ref[...] = vsel

    sel = rows[TOPK - 1]
    for k in range(TOPK - 2, -1, -1):
        sel = jnp.where(v4[:, k:k + 1] == vsel, rows[k], sel)
    dq = zn - sel
    part_loss = jnp.sum(dq * dq) * (GAMMA / (N * D))

    ohv = jnp.where(vsel == iota, 1.0, 0.0)
    part_counts = jnp.sum(ohv, axis=0, keepdims=True)

    @pl.when(b == 0)
    def _init():
        counts_ref[...] = part_counts
        loss_ref[0, 0] = part_loss

    @pl.when(b > 0)
    def _acc():
        counts_ref[...] += part_counts
        loss_ref[0, 0] += part_loss

    @pl.when(b == B - 1)
    def _fin():
        p = counts_ref[...] * (1.0 / N)
        ent = jnp.sum(p * jnp.log(p + 1e-7))
        cpx_ref[0, 0] = jnp.exp(-ent)


def _sc_gather_body(table_hbm, idx_hbm, out_hbm, idx_v, rows_v, sem):
    wid = lax.axis_index("s") * SC_NC + lax.axis_index("c")
    base = wid * GB_PER_W
    pltpu.sync_copy(idx_hbm.at[pl.ds(base, GB_PER_W)], idx_v)
    pltpu.async_copy(table_hbm.at[idx_v], rows_v, sem).wait()
    pltpu.sync_copy(rows_v, out_hbm.at[pl.ds(base, GB_PER_W)])


@functools.lru_cache(maxsize=1)
def _sc_gather():
    return pl.kernel(
        _sc_gather_body,
        mesh=plsc.VectorSubcoreMesh(core_axis_name="c", subcore_axis_name="s"),
        out_type=jax.ShapeDtypeStruct((N, D), jnp.float32),
        scratch_types=[
            pltpu.VMEM((GB_PER_W,), jnp.int32),
            pltpu.VMEM((GB_PER_W, D), jnp.float32),
            pltpu.SemaphoreType.DMA,
        ],
    )


def kernel(x, embedding, conv_w, gn_w, gn_b):
    emb = embedding.reshape(V, D)
    gw = gn_w.reshape(1, D)
    gb = gn_b.reshape(1, D)

    probs, vsel, _counts, cpx, loss = pl.pallas_call(
        _fused_body,
        grid=(B,),
        in_specs=[
            pl.BlockSpec((None, T, D), lambda b: (b, 0, 0)),
            pl.BlockSpec((D, D), lambda b: (0, 0)),
            pl.BlockSpec((1, D), lambda b: (0, 0)),
            pl.BlockSpec((1, D), lambda b: (0, 0)),
            pl.BlockSpec((V, D), lambda b: (0, 0)),
        ],
        out_specs=[
            pl.BlockSpec((None, T, V), lambda b: (b, 0, 0)),
            pl.BlockSpec((None, T, 1), lambda b: (b, 0, 0)),
            pl.BlockSpec((1, V), lambda b: (0, 0)),
            pl.BlockSpec(memory_space=pltpu.SMEM),
            pl.BlockSpec(memory_space=pltpu.SMEM),
        ],
        out_shape=[
            jax.ShapeDtypeStruct((B, T, V), jnp.float32),
            jax.ShapeDtypeStruct((B, T, 1), jnp.int32),
            jax.ShapeDtypeStruct((1, V), jnp.float32),
            jax.ShapeDtypeStruct((1, 1), jnp.float32),
            jax.ShapeDtypeStruct((1, 1), jnp.float32),
        ],
    )(x, conv_w, gw, gb, emb)

    out2 = _sc_gather()(emb, vsel.reshape(N))
    out = out2.reshape(B, T, D)
    return out, probs, cpx[0, 0], loss[0, 0]


# speedup vs baseline: 1.0581x; 1.0581x over previous
import functools

import jax
import jax.numpy as jnp
from jax import lax
from jax.experimental import pallas as pl
from jax.experimental.pallas import tpu as pltpu
from jax.experimental.pallas import tpu_sc as plsc

V = 512
D = 128
B = 8
T = 256
N = B * T
TOPK = 3
GAMMA = 0.25

SC_NC = 2
SC_NS = 16
SC_NW = SC_NC * SC_NS
GB_PER_W = N // SC_NW


def _fused_body(x_ref, cw_ref, gw_ref, gb_ref, emb_ref,
                probs_ref, vsel_ref, counts_ref, cpx_ref, loss_ref):
    b = pl.program_id(0)
    xb = x_ref[...]
    ze = lax.dot_general(xb, cw_ref[...], (((1,), (1,)), ((), ())),
                         preferred_element_type=jnp.float32)
    mean = jnp.mean(ze)
    var = jnp.mean((ze - mean) ** 2)
    zn = (ze - mean) * lax.rsqrt(var + 1e-5)
    zn = zn * gw_ref[...] + gb_ref[...]

    emb = emb_ref[...]
    g = lax.dot_general(zn, emb, (((1,), (1,)), ((), ())),
                        preferred_element_type=jnp.float32)
    e2 = emb * emb
    esq = lax.dot_general(jnp.ones((1, D), jnp.float32), e2,
                          (((1,), (1,)), ((), ())),
                          preferred_element_type=jnp.float32)
    zsq = jnp.sum(zn * zn, axis=1, keepdims=True)
    d2 = jnp.maximum(zsq - 2.0 * g + esq, 0.0)

    dmm = jnp.sqrt(d2)
    m = jnp.max(-dmm, axis=1, keepdims=True)
    e = jnp.exp(-dmm - m)
    probs_ref[...] = e / jnp.sum(e, axis=1, keepdims=True)

    iota = lax.broadcasted_iota(jnp.int32, (T, V), 1)
    cur = d2
    dists = []
    rows = []
    picks = []
    for _ in range(TOPK):
        mn = jnp.min(cur, axis=1, keepdims=True)
        ik = jnp.min(jnp.where(cur == mn, iota, 1 << 20), axis=1,
                     keepdims=True)
        sel_mask = iota == ik
        cur = jnp.where(sel_mask, 1e30, cur)
        oh = jnp.where(sel_mask, 1.0, 0.0)
        row = lax.dot_general(oh, emb, (((1,), (0,)), ((), ())),
                              precision=lax.Precision.HIGHEST,
                              preferred_element_type=jnp.float32)
        diff = zn - row
        s = jnp.sum(diff * diff, axis=1, keepdims=True)
        picks.append(ik)
        rows.append(row)
        dists.append(jnp.sqrt(s))
    d4 = jnp.concatenate(dists, axis=1)
    v4 = jnp.concatenate(picks, axis=1)
    best = jnp.min(d4, axis=1, keepdims=True)
    vsel = jnp.min(jnp.where(d4 == best, v4, 1 << 20), axis=1,
                   keepdims=True)
    vsel_ref[...] = vsel

    sel = rows[TOPK - 1]
    for k in range(TOPK - 2, -1, -1):
        sel = jnp.where(v4[:, k:k + 1] == vsel, rows[k], sel)
    dq = zn - sel
    part_loss = jnp.sum(dq * dq) * (GAMMA / (N * D))

    ohv = jnp.where(vsel == iota, 1.0, 0.0)
    part_counts = jnp.sum(ohv, axis=0, keepdims=True)

    @pl.when(b == 0)
    def _init():
        counts_ref[...] = part_counts
        loss_ref[0, 0] = part_loss

    @pl.when(b > 0)
    def _acc():
        counts_ref[...] += part_counts
        loss_ref[0, 0] += part_loss

    @pl.when(b == B - 1)
    def _fin():
        p = counts_ref[...] * (1.0 / N)
        ent = jnp.sum(p * jnp.log(p + 1e-7))
        cpx_ref[0, 0] = jnp.exp(-ent)


def _sc_gather_body(table_hbm, idx_hbm, out_hbm, idx_v, rows_v, sem):
    wid = lax.axis_index("s") * SC_NC + lax.axis_index("c")
    base = wid * GB_PER_W
    pltpu.sync_copy(idx_hbm.at[pl.ds(base, GB_PER_W)], idx_v)
    pltpu.async_copy(table_hbm.at[idx_v], rows_v, sem).wait()
    pltpu.sync_copy(rows_v, out_hbm.at[pl.ds(base, GB_PER_W)])


@functools.lru_cache(maxsize=1)
def _sc_gather():
    return pl.kernel(
        _sc_gather_body,
        mesh=plsc.VectorSubcoreMesh(core_axis_name="c", subcore_axis_name="s"),
        out_type=jax.ShapeDtypeStruct((N, D), jnp.float32),
        scratch_types=[
            pltpu.VMEM((GB_PER_W,), jnp.int32),
            pltpu.VMEM((GB_PER_W, D), jnp.float32),
            pltpu.SemaphoreType.DMA,
        ],
    )


def kernel(x, embedding, conv_w, gn_w, gn_b):
    emb = embedding.reshape(V, D)
    gw = gn_w.reshape(1, D)
    gb = gn_b.reshape(1, D)

    probs, vsel, _counts, cpx, loss = pl.pallas_call(
        _fused_body,
        grid=(B,),
        in_specs=[
            pl.BlockSpec((None, T, D), lambda b: (b, 0, 0)),
            pl.BlockSpec((D, D), lambda b: (0, 0)),
            pl.BlockSpec((1, D), lambda b: (0, 0)),
            pl.BlockSpec((1, D), lambda b: (0, 0)),
            pl.BlockSpec((V, D), lambda b: (0, 0)),
        ],
        out_specs=[
            pl.BlockSpec((None, T, V), lambda b: (b, 0, 0)),
            pl.BlockSpec((None, T, 1), lambda b: (b, 0, 0)),
            pl.BlockSpec((1, V), lambda b: (0, 0)),
            pl.BlockSpec(memory_space=pltpu.SMEM),
            pl.BlockSpec(memory_space=pltpu.SMEM),
        ],
        out_shape=[
            jax.ShapeDtypeStruct((B, T, V), jnp.float32),
            jax.ShapeDtypeStruct((B, T, 1), jnp.int32),
            jax.ShapeDtypeStruct((1, V), jnp.float32),
            jax.ShapeDtypeStruct((1, 1), jnp.float32),
            jax.ShapeDtypeStruct((1, 1), jnp.float32),
        ],
    )(x, conv_w, gw, gb, emb)

    out2 = _sc_gather()(emb, vsel.reshape(N))
    out = out2.reshape(B, T, D)
    return out, probs, cpx[0, 0], loss[0, 0]
